# baseline (speedup 1.0000x reference)
import jax
import jax.numpy as jnp
from jax import lax
from jax.experimental import pallas as pl
from jax.experimental.pallas import tpu as pltpu

_H = 768
_E = 64
_K = 8
_F = 768
_SF = 2048
_T = 256
_TM = 64


def _sig(v):
    return 1.0 / (1.0 + jnp.exp(-v))


def _prologue_body(x_ref, gw_ref, swg_ref, swu_ref, swd_ref, sgw_ref,
                   meta_ref, rm_ref, cb_ref, sh_ref):
    x = x_ref[...]
    lt = lax.dot_general(gw_ref[...], x, (((1,), (1,)), ((), ())),
                         preferred_element_type=jnp.float32)
    m = jnp.max(lt, axis=0, keepdims=True)
    p = jnp.exp(lt - m)
    probs = p / jnp.sum(p, axis=0, keepdims=True)
    eidx = lax.broadcasted_iota(jnp.int32, (_E, _T), 0).astype(jnp.float32)
    work = probs
    maskf = jnp.zeros((_E, _T), jnp.float32)
    for _ in range(_K):
        mx = jnp.max(work, axis=0, keepdims=True)
        cand = jnp.where(work == mx, eidx, float(_E))
        jmin = jnp.min(cand, axis=0, keepdims=True)
        oh = (eidx == jmin).astype(jnp.float32)
        maskf = maskf + oh
        work = jnp.where(oh > 0, -1.0, work)
    comb = maskf * probs
    ta = lax.broadcasted_iota(jnp.int32, (_T, _T), 0)
    tb = lax.broadcasted_iota(jnp.int32, (_T, _T), 1)
    tri = (ta <= tb).astype(jnp.float32)
    ranks = jnp.dot(maskf, tri, preferred_element_type=jnp.float32)
    rm = jnp.where(maskf > 0, ranks, 0.0)
    rm_ref[...] = rm
    cb_ref[...] = comb
    counts = jnp.sum(maskf, axis=1, keepdims=True)
    nb = jnp.floor((counts + (_TM - 1)) / _TM)
    lane = lax.broadcasted_iota(jnp.int32, (_E, 8), 1)
    nb8 = jnp.broadcast_to(nb, (_E, 8))
    meta_ref[...] = jnp.where(lane == 0, nb8, 0.0).astype(jnp.int32)
    sg = jnp.dot(x, swg_ref[...], preferred_element_type=jnp.float32)
    su = jnp.dot(x, swu_ref[...], preferred_element_type=jnp.float32)
    sh = (sg * _sig(sg)) * su
    so = jnp.dot(sh, swd_ref[...], preferred_element_type=jnp.float32)
    gate = jnp.sum(x * jnp.broadcast_to(sgw_ref[...], (_T, _H)),
                   axis=1, keepdims=True)
    sh_ref[...] = _sig(gate) * so


def _moe_body(meta_ref, x_ref, sh_ref, rm_ref, cb_ref, wg_ref, wu_ref, wd_ref,
              out_ref):
    e = pl.program_id(0)

    @pl.when(e == 0)
    def _():
        out_ref[...] = sh_ref[...]

    nblk = meta_ref[e, 0]
    rm = jnp.broadcast_to(rm_ref[pl.ds(e, 1), :], (_TM, _T))
    cb = jnp.broadcast_to(cb_ref[pl.ds(e, 1), :], (_TM, _T))
    x = x_ref[...]
    wg = wg_ref[0]
    wu = wu_ref[0]
    wd = wd_ref[0]

    riota = lax.broadcasted_iota(jnp.int32, (_TM, _T), 0).astype(jnp.float32)

    for j in range(_T // _TM):
        @pl.when(j < nblk)
        def _(j=j):
            pos = float(j * _TM + 1) + riota
            P = (rm == pos).astype(jnp.float32)
            X = jnp.dot(P, x, preferred_element_type=jnp.float32)
            g = jnp.dot(X, wg, preferred_element_type=jnp.float32)
            u = jnp.dot(X, wu, preferred_element_type=jnp.float32)
            h = (g * _sig(g)) * u
            o = jnp.dot(h, wd, preferred_element_type=jnp.float32)
            w_row = jnp.sum(P * cb, axis=1, keepdims=True)
            contrib = lax.dot_general(P, o * w_row,
                                      (((0,), (0,)), ((), ())),
                                      preferred_element_type=jnp.float32)
            out_ref[...] += contrib


def kernel(x, gate_w, w_gate, w_up, w_down, sw_gate, sw_up, sw_down,
           shared_gate_w):
    meta, rm, cb, shared = pl.pallas_call(
        _prologue_body,
        out_shape=(
            jax.ShapeDtypeStruct((_E, 8), jnp.int32),
            jax.ShapeDtypeStruct((_E, _T), jnp.float32),
            jax.ShapeDtypeStruct((_E, _T), jnp.float32),
            jax.ShapeDtypeStruct((_T, _H), jnp.float32),
        ),
    )(x, gate_w, sw_gate, sw_up, sw_down, shared_gate_w)

    grid_spec = pltpu.PrefetchScalarGridSpec(
        num_scalar_prefetch=1,
        grid=(_E,),
        in_specs=[
            pl.BlockSpec((_T, _H), lambda e, m: (0, 0)),
            pl.BlockSpec((_T, _H), lambda e, m: (0, 0)),
            pl.BlockSpec((_E, _T), lambda e, m: (0, 0)),
            pl.BlockSpec((_E, _T), lambda e, m: (0, 0)),
            pl.BlockSpec((1, _H, _F), lambda e, m: (e, 0, 0)),
            pl.BlockSpec((1, _H, _F), lambda e, m: (e, 0, 0)),
            pl.BlockSpec((1, _F, _H), lambda e, m: (e, 0, 0)),
        ],
        out_specs=pl.BlockSpec((_T, _H), lambda e, m: (0, 0)),
    )
    out = pl.pallas_call(
        _moe_body,
        grid_spec=grid_spec,
        out_shape=jax.ShapeDtypeStruct((_T, _H), jnp.float32),
        compiler_params=pltpu.CompilerParams(
            dimension_semantics=("arbitrary",)),
    )(meta, x, shared, rm, cb, w_gate, w_up, w_down)
    return out


# speedup vs baseline: 1.1009x; 1.1009x over previous
import jax
import jax.numpy as jnp
from jax import lax
from jax.experimental import pallas as pl
from jax.experimental.pallas import tpu as pltpu

_H = 768
_E = 64
_K = 8
_F = 768
_SF = 2048
_T = 256
_TM = 64


def _sig(v):
    return 1.0 / (1.0 + jnp.exp(-v))


def _prologue_body(x_ref, gw_ref, swg_ref, swu_ref, swd_ref, sgw_ref,
                   meta_ref, rm_ref, cb_ref, sh_ref):
    x = x_ref[...]
    lt = lax.dot_general(gw_ref[...], x, (((1,), (1,)), ((), ())),
                         preferred_element_type=jnp.float32)
    m = jnp.max(lt, axis=0, keepdims=True)
    p = jnp.exp(lt - m)
    probs = p / jnp.sum(p, axis=0, keepdims=True)
    eidx = lax.broadcasted_iota(jnp.int32, (_E, _T), 0).astype(jnp.float32)
    work = probs
    maskf = jnp.zeros((_E, _T), jnp.float32)
    for _ in range(_K):
        mx = jnp.max(work, axis=0, keepdims=True)
        cand = jnp.where(work == mx, eidx, float(_E))
        jmin = jnp.min(cand, axis=0, keepdims=True)
        oh = (eidx == jmin).astype(jnp.float32)
        maskf = maskf + oh
        work = jnp.where(oh > 0, -1.0, work)
    comb = maskf * probs
    ta = lax.broadcasted_iota(jnp.int32, (_T, _T), 0)
    tb = lax.broadcasted_iota(jnp.int32, (_T, _T), 1)
    tri = (ta <= tb).astype(jnp.float32)
    ranks = jnp.dot(maskf, tri, preferred_element_type=jnp.float32)
    rm = jnp.where(maskf > 0, ranks, 0.0)
    rm_ref[...] = rm
    cb_ref[...] = comb
    counts = jnp.sum(maskf, axis=1, keepdims=True)
    nb = jnp.floor((counts + (_TM - 1)) / _TM)
    lane = lax.broadcasted_iota(jnp.int32, (_E, 8), 1)
    nb8 = jnp.broadcast_to(nb, (_E, 8))
    meta_ref[...] = jnp.where(lane == 0, nb8, 0.0).astype(jnp.int32)
    sg = jnp.dot(x, swg_ref[...], preferred_element_type=jnp.float32)
    su = jnp.dot(x, swu_ref[...], preferred_element_type=jnp.float32)
    sh = (sg * _sig(sg)) * su
    so = jnp.dot(sh, swd_ref[...], preferred_element_type=jnp.float32)
    gate = jnp.sum(x * jnp.broadcast_to(sgw_ref[...], (_T, _H)),
                   axis=1, keepdims=True)
    sh_ref[...] = _sig(gate) * so


def _moe_body(meta_ref, x_ref, sh_ref, rm_ref, cb_ref, wg_ref, wu_ref, wd_ref,
              out_ref):
    e = pl.program_id(0)

    @pl.when(e == 0)
    def _():
        out_ref[...] = sh_ref[...]

    nblk = meta_ref[e, 0]

    for j in range(_T // _TM):
        @pl.when(j < nblk)
        def _(j=j):
            rm = jnp.broadcast_to(rm_ref[pl.ds(e, 1), :], (_TM, _T))
            cb = jnp.broadcast_to(cb_ref[pl.ds(e, 1), :], (_TM, _T))
            pos = float(j * _TM + 1) + \
                lax.broadcasted_iota(jnp.int32, (_TM, _T), 0).astype(
                    jnp.float32)
            P = (rm == pos).astype(jnp.float32)
            X = jnp.dot(P, x_ref[...], preferred_element_type=jnp.float32)
            g = jnp.dot(X, wg_ref[0], preferred_element_type=jnp.float32)
            u = jnp.dot(X, wu_ref[0], preferred_element_type=jnp.float32)
            h = (g * _sig(g)) * u
            o = jnp.dot(h, wd_ref[0], preferred_element_type=jnp.float32)
            w_row = jnp.sum(P * cb, axis=1, keepdims=True)
            contrib = lax.dot_general(P, o * w_row,
                                      (((0,), (0,)), ((), ())),
                                      preferred_element_type=jnp.float32)
            out_ref[...] += contrib


def kernel(x, gate_w, w_gate, w_up, w_down, sw_gate, sw_up, sw_down,
           shared_gate_w):
    meta, rm, cb, shared = pl.pallas_call(
        _prologue_body,
        out_shape=(
            jax.ShapeDtypeStruct((_E, 8), jnp.int32),
            jax.ShapeDtypeStruct((_E, _T), jnp.float32),
            jax.ShapeDtypeStruct((_E, _T), jnp.float32),
            jax.ShapeDtypeStruct((_T, _H), jnp.float32),
        ),
    )(x, gate_w, sw_gate, sw_up, sw_down, shared_gate_w)

    grid_spec = pltpu.PrefetchScalarGridSpec(
        num_scalar_prefetch=1,
        grid=(_E,),
        in_specs=[
            pl.BlockSpec((_T, _H), lambda e, m: (0, 0)),
            pl.BlockSpec((_T, _H), lambda e, m: (0, 0)),
            pl.BlockSpec((_E, _T), lambda e, m: (0, 0)),
            pl.BlockSpec((_E, _T), lambda e, m: (0, 0)),
            pl.BlockSpec((1, _H, _F), lambda e, m: (e, 0, 0)),
            pl.BlockSpec((1, _H, _F), lambda e, m: (e, 0, 0)),
            pl.BlockSpec((1, _F, _H), lambda e, m: (e, 0, 0)),
        ],
        out_specs=pl.BlockSpec((_T, _H), lambda e, m: (0, 0)),
    )
    out = pl.pallas_call(
        _moe_body,
        grid_spec=grid_spec,
        out_shape=jax.ShapeDtypeStruct((_T, _H), jnp.float32),
        compiler_params=pltpu.CompilerParams(
            dimension_semantics=("arbitrary",)),
    )(meta, x, shared, rm, cb, w_gate, w_up, w_down)
    return out


# speedup vs baseline: 1.1037x; 1.0025x over previous
import jax
import jax.numpy as jnp
from jax import lax
from jax.experimental import pallas as pl
from jax.experimental.pallas import tpu as pltpu

_H = 768
_E = 64
_K = 8
_F = 768
_SF = 2048
_T = 256
_TM = 64


def _sig(v):
    return 1.0 / (1.0 + jnp.exp(-v))


def _prologue_body(x_ref, gw_ref, swg_ref, swu_ref, swd_ref, sgw_ref,
                   meta_ref, rm_ref, cb_ref, sh_ref):
    x = x_ref[...]
    lt = lax.dot_general(gw_ref[...], x, (((1,), (1,)), ((), ())),
                         preferred_element_type=jnp.float32)
    m = jnp.max(lt, axis=0, keepdims=True)
    p = jnp.exp(lt - m)
    probs = p / jnp.sum(p, axis=0, keepdims=True)
    eidx = lax.broadcasted_iota(jnp.int32, (_E, _T), 0).astype(jnp.float32)
    work = probs
    maskf = jnp.zeros((_E, _T), jnp.float32)
    for _ in range(_K):
        mx = jnp.max(work, axis=0, keepdims=True)
        cand = jnp.where(work == mx, eidx, float(_E))
        jmin = jnp.min(cand, axis=0, keepdims=True)
        oh = (eidx == jmin).astype(jnp.float32)
        maskf = maskf + oh
        work = jnp.where(oh > 0, -1.0, work)
    comb = maskf * probs
    ta = lax.broadcasted_iota(jnp.int32, (_T, _T), 0)
    tb = lax.broadcasted_iota(jnp.int32, (_T, _T), 1)
    tri = (ta <= tb).astype(jnp.float32)
    ranks = jnp.dot(maskf, tri, preferred_element_type=jnp.float32)
    rm = jnp.where(maskf > 0, ranks, 0.0)
    rm_ref[...] = rm
    cb_ref[...] = comb
    counts = jnp.sum(maskf, axis=1, keepdims=True)
    nb = jnp.floor((counts + (_TM - 1)) / _TM)
    lane = lax.broadcasted_iota(jnp.int32, (_E, 8), 1)
    nb8 = jnp.broadcast_to(nb, (_E, 8))
    meta_ref[...] = jnp.where(lane == 0, nb8, 0.0).astype(jnp.int32)
    sg = jnp.dot(x, swg_ref[...], preferred_element_type=jnp.float32)
    su = jnp.dot(x, swu_ref[...], preferred_element_type=jnp.float32)
    sh = (sg * _sig(sg)) * su
    so = jnp.dot(sh, swd_ref[...], preferred_element_type=jnp.float32)
    gate = jnp.sum(x * jnp.broadcast_to(sgw_ref[...], (_T, _H)),
                   axis=1, keepdims=True)
    sh_ref[...] = _sig(gate) * so


def _moe_body(meta_ref, x_ref, sh_ref, rm_ref, cb_ref, wg_ref, wu_ref, wd_ref,
              out_ref):
    e = pl.program_id(0)

    @pl.when(e == 0)
    def _():
        out_ref[...] = sh_ref[...]

    nblk = meta_ref[e, 0]

    for j in range(_T // _TM):
        @pl.when(j < nblk)
        def _(j=j):
            rm = jnp.broadcast_to(rm_ref[pl.ds(e, 1), :], (_TM, _T))
            cb = jnp.broadcast_to(cb_ref[pl.ds(e, 1), :], (_TM, _T))
            pos = float(j * _TM + 1) + \
                lax.broadcasted_iota(jnp.int32, (_TM, _T), 0).astype(
                    jnp.float32)
            P = (rm == pos).astype(jnp.float32)
            X = jnp.dot(P, x_ref[...], preferred_element_type=jnp.float32)
            g = jnp.dot(X, wg_ref[0], preferred_element_type=jnp.float32,
                        precision=lax.Precision.DEFAULT)
            u = jnp.dot(X, wu_ref[0], preferred_element_type=jnp.float32,
                        precision=lax.Precision.DEFAULT)
            h = (g * _sig(g)) * u
            o = jnp.dot(h, wd_ref[0], preferred_element_type=jnp.float32,
                        precision=lax.Precision.DEFAULT)
            contrib = lax.dot_general(P * cb, o,
                                      (((0,), (0,)), ((), ())),
                                      preferred_element_type=jnp.float32)
            out_ref[...] += contrib


def kernel(x, gate_w, w_gate, w_up, w_down, sw_gate, sw_up, sw_down,
           shared_gate_w):
    meta, rm, cb, shared = pl.pallas_call(
        _prologue_body,
        out_shape=(
            jax.ShapeDtypeStruct((_E, 8), jnp.int32),
            jax.ShapeDtypeStruct((_E, _T), jnp.float32),
            jax.ShapeDtypeStruct((_E, _T), jnp.float32),
            jax.ShapeDtypeStruct((_T, _H), jnp.float32),
        ),
    )(x, gate_w, sw_gate, sw_up, sw_down, shared_gate_w)

    grid_spec = pltpu.PrefetchScalarGridSpec(
        num_scalar_prefetch=1,
        grid=(_E,),
        in_specs=[
            pl.BlockSpec((_T, _H), lambda e, m: (0, 0)),
            pl.BlockSpec((_T, _H), lambda e, m: (0, 0)),
            pl.BlockSpec((_E, _T), lambda e, m: (0, 0)),
            pl.BlockSpec((_E, _T), lambda e, m: (0, 0)),
            pl.BlockSpec((1, _H, _F), lambda e, m: (e, 0, 0)),
            pl.BlockSpec((1, _H, _F), lambda e, m: (e, 0, 0)),
            pl.BlockSpec((1, _F, _H), lambda e, m: (e, 0, 0)),
        ],
        out_specs=pl.BlockSpec((_T, _H), lambda e, m: (0, 0)),
    )
    out = pl.pallas_call(
        _moe_body,
        grid_spec=grid_spec,
        out_shape=jax.ShapeDtypeStruct((_T, _H), jnp.float32),
        compiler_params=pltpu.CompilerParams(
            dimension_semantics=("arbitrary",)),
    )(meta, x, shared, rm, cb, w_gate, w_up, w_down)
    return out


# speedup vs baseline: 1.1778x; 1.0671x over previous
import jax
import jax.numpy as jnp
from jax import lax
from jax.experimental import pallas as pl
from jax.experimental.pallas import tpu as pltpu

_H = 768
_E = 64
_K = 8
_F = 768
_SF = 2048
_T = 256
_TM = 64
_NBUF = 3


def _sig(v):
    return 1.0 / (1.0 + jnp.exp(-v))


def _prologue_body(x_ref, gw_ref, swg_ref, swu_ref, swd_ref, sgw_ref,
                   meta_ref, rm_ref, cb_ref, sh_ref):
    x = x_ref[...]
    lt = lax.dot_general(gw_ref[...], x, (((1,), (1,)), ((), ())),
                         preferred_element_type=jnp.float32)
    m = jnp.max(lt, axis=0, keepdims=True)
    p = jnp.exp(lt - m)
    probs = p / jnp.sum(p, axis=0, keepdims=True)
    eidx = lax.broadcasted_iota(jnp.int32, (_E, _T), 0).astype(jnp.float32)
    work = probs
    maskf = jnp.zeros((_E, _T), jnp.float32)
    for _ in range(_K):
        mx = jnp.max(work, axis=0, keepdims=True)
        cand = jnp.where(work == mx, eidx, float(_E))
        jmin = jnp.min(cand, axis=0, keepdims=True)
        oh = (eidx == jmin).astype(jnp.float32)
        maskf = maskf + oh
        work = jnp.where(oh > 0, -1.0, work)
    comb = maskf * probs
    ta = lax.broadcasted_iota(jnp.int32, (_T, _T), 0)
    tb = lax.broadcasted_iota(jnp.int32, (_T, _T), 1)
    tri = (ta <= tb).astype(jnp.float32)
    ranks = jnp.dot(maskf, tri, preferred_element_type=jnp.float32)
    rm = jnp.where(maskf > 0, ranks, 0.0)
    rm_ref[...] = rm
    cb_ref[...] = comb
    counts = jnp.sum(maskf, axis=1, keepdims=True)
    nb = jnp.floor((counts + (_TM - 1)) / _TM)
    lane = lax.broadcasted_iota(jnp.int32, (_E, 8), 1)
    nb8 = jnp.broadcast_to(nb, (_E, 8))
    meta_ref[...] = jnp.where(lane == 0, nb8, 0.0).astype(jnp.int32)
    sg = jnp.dot(x, swg_ref[...], preferred_element_type=jnp.float32)
    su = jnp.dot(x, swu_ref[...], preferred_element_type=jnp.float32)
    sh = (sg * _sig(sg)) * su
    so = jnp.dot(sh, swd_ref[...], preferred_element_type=jnp.float32)
    gate = jnp.sum(x * jnp.broadcast_to(sgw_ref[...], (_T, _H)),
                   axis=1, keepdims=True)
    sh_ref[...] = _sig(gate) * so


def _moe_body(meta_ref, x_ref, sh_ref, rm_ref, cb_ref, wg_ref, wu_ref, wd_ref,
              out_ref, wgb, wub, wdb, sems):
    e = pl.program_id(0)

    def issue(idx):
        slot = lax.rem(idx, _NBUF)
        pltpu.make_async_copy(wg_ref.at[idx], wgb.at[slot],
                              sems.at[slot, 0]).start()
        pltpu.make_async_copy(wu_ref.at[idx], wub.at[slot],
                              sems.at[slot, 1]).start()
        pltpu.make_async_copy(wd_ref.at[idx], wdb.at[slot],
                              sems.at[slot, 2]).start()

    @pl.when(e == 0)
    def _():
        out_ref[...] = sh_ref[...]
        issue(0)
        issue(1)

    @pl.when(e + 2 < _E)
    def _():
        issue(e + 2)

    slot = lax.rem(e, _NBUF)
    pltpu.make_async_copy(wg_ref.at[e], wgb.at[slot], sems.at[slot, 0]).wait()
    pltpu.make_async_copy(wu_ref.at[e], wub.at[slot], sems.at[slot, 1]).wait()
    pltpu.make_async_copy(wd_ref.at[e], wdb.at[slot], sems.at[slot, 2]).wait()

    nblk = meta_ref[e, 0]

    for j in range(_T // _TM):
        @pl.when(j < nblk)
        def _(j=j):
            rm = jnp.broadcast_to(rm_ref[pl.ds(e, 1), :], (_TM, _T))
            cb = jnp.broadcast_to(cb_ref[pl.ds(e, 1), :], (_TM, _T))
            pos = float(j * _TM + 1) + \
                lax.broadcasted_iota(jnp.int32, (_TM, _T), 0).astype(
                    jnp.float32)
            P = (rm == pos).astype(jnp.float32)
            X = jnp.dot(P, x_ref[...], preferred_element_type=jnp.float32)
            wg = wgb[pl.ds(slot, 1)][0]
            wu = wub[pl.ds(slot, 1)][0]
            wd = wdb[pl.ds(slot, 1)][0]
            g = jnp.dot(X, wg, preferred_element_type=jnp.float32,
                        precision=lax.Precision.DEFAULT)
            u = jnp.dot(X, wu, preferred_element_type=jnp.float32,
                        precision=lax.Precision.DEFAULT)
            h = (g * _sig(g)) * u
            o = jnp.dot(h, wd, preferred_element_type=jnp.float32,
                        precision=lax.Precision.DEFAULT)
            contrib = lax.dot_general(P * cb, o,
                                      (((0,), (0,)), ((), ())),
                                      preferred_element_type=jnp.float32)
            out_ref[...] += contrib


def kernel(x, gate_w, w_gate, w_up, w_down, sw_gate, sw_up, sw_down,
           shared_gate_w):
    meta, rm, cb, shared = pl.pallas_call(
        _prologue_body,
        out_shape=(
            jax.ShapeDtypeStruct((_E, 8), jnp.int32),
            jax.ShapeDtypeStruct((_E, _T), jnp.float32),
            jax.ShapeDtypeStruct((_E, _T), jnp.float32),
            jax.ShapeDtypeStruct((_T, _H), jnp.float32),
        ),
    )(x, gate_w, sw_gate, sw_up, sw_down, shared_gate_w)

    grid_spec = pltpu.PrefetchScalarGridSpec(
        num_scalar_prefetch=1,
        grid=(_E,),
        in_specs=[
            pl.BlockSpec((_T, _H), lambda e, m: (0, 0)),
            pl.BlockSpec((_T, _H), lambda e, m: (0, 0)),
            pl.BlockSpec((_E, _T), lambda e, m: (0, 0)),
            pl.BlockSpec((_E, _T), lambda e, m: (0, 0)),
            pl.BlockSpec(memory_space=pl.ANY),
            pl.BlockSpec(memory_space=pl.ANY),
            pl.BlockSpec(memory_space=pl.ANY),
        ],
        out_specs=pl.BlockSpec((_T, _H), lambda e, m: (0, 0)),
        scratch_shapes=[
            pltpu.VMEM((_NBUF, _H, _F), jnp.float32),
            pltpu.VMEM((_NBUF, _H, _F), jnp.float32),
            pltpu.VMEM((_NBUF, _F, _H), jnp.float32),
            pltpu.SemaphoreType.DMA((_NBUF, 3)),
        ],
    )
    out = pl.pallas_call(
        _moe_body,
        grid_spec=grid_spec,
        out_shape=jax.ShapeDtypeStruct((_T, _H), jnp.float32),
        compiler_params=pltpu.CompilerParams(
            dimension_semantics=("arbitrary",)),
    )(meta, x, shared, rm, cb, w_gate, w_up, w_down)
    return out
